# baseline (speedup 1.0000x reference)
import functools

import jax
import jax.numpy as jnp
from jax import lax
from jax.experimental import pallas as pl
from jax.experimental.pallas import tpu as pltpu
from jax.experimental.pallas import tpu_sc as plsc

NC = 2
NS = 16
NW = NC * NS

LANES = 16
BB = 128


@functools.lru_cache(maxsize=None)
def _build(b, l, vocab, embed):
    nbb = b // BB
    bb_per_w = nbb // NW
    b_per_w = b // NW
    nrb = embed // 8
    units = l * bb_per_w

    mesh = plsc.VectorSubcoreMesh(core_axis_name="c", subcore_axis_name="s")

    @functools.partial(
        pl.kernel,
        out_type=jax.ShapeDtypeStruct((l, nrb, nbb, 8, BB), jnp.float32),
        mesh=mesh,
        scratch_types=[
            pltpu.VMEM((l, b_per_w), jnp.int32),
            pltpu.VMEM((2, BB, embed), jnp.float32),
            pltpu.VMEM((2, nrb, 8, BB), jnp.float32),
            pltpu.SemaphoreType.DMA,
            pltpu.SemaphoreType.DMA,
            pltpu.SemaphoreType.DMA,
            pltpu.SemaphoreType.DMA,
        ],
        compiler_params=pltpu.CompilerParams(use_tc_tiling_on_sc=False, needs_layout_passes=False),
    )
    def gather_kernel(idx_hbm, table_hbm, out_hbm, idx_v, rows_v, stg_v,
                      gsem0, gsem1, ssem0, ssem1):
        wid = lax.axis_index("s") * NC + lax.axis_index("c")
        row16s = [lax.iota(jnp.int32, LANES) + m * LANES
                  for m in range(BB // LANES)]
        pltpu.sync_copy(idx_hbm.at[:, pl.ds(wid * b_per_w, b_per_w)], idx_v)

        gsems = (gsem0, gsem1)
        ssems = (ssem0, ssem1)

        def unit_lcb(u):
            return u // bb_per_w, lax.rem(u, bb_per_w)

        def gather_descr(u, bufset):
            ul, ucb = unit_lcb(u)
            return pltpu.make_async_copy(
                table_hbm.at[idx_v.at[ul, pl.ds(ucb * BB, BB)]],
                rows_v.at[bufset],
                gsems[bufset],
            )

        def store_descr(u, bufset, rb):
            ul, ucb = unit_lcb(u)
            return pltpu.make_async_copy(
                stg_v.at[bufset, rb],
                out_hbm.at[ul, rb, wid * bb_per_w + ucb],
                ssems[bufset],
            )

        def transpose_unit(bufset):
            src = rows_v.at[bufset]
            for rb in range(nrb):
                for s in range(8):
                    col16 = jnp.full((LANES,), rb * 8 + s, jnp.int32)
                    for m in range(BB // LANES):
                        val = plsc.load_gather(src, [row16s[m], col16])
                        stg_v[bufset, rb, s, pl.ds(m * LANES, LANES)] = val

        gather_descr(0, 0).start()

        def body(i, _):
            for half in range(2):
                u = 2 * i + half
                bufset = half

                @pl.when(u + 1 < units)
                def _():
                    @pl.when(u >= 1)
                    def _():
                        for rb in range(nrb):
                            store_descr(u - 1, 1 - bufset, rb).wait()

                    gather_descr(u + 1, 1 - bufset).start()

                gather_descr(u, bufset).wait()
                transpose_unit(bufset)
                for rb in range(nrb):
                    store_descr(u, bufset, rb).start()
            return 0

        lax.fori_loop(0, units // 2, body, 0)
        for rb in range(nrb):
            store_descr(units - 2, 0, rb).wait()
            store_descr(units - 1, 1, rb).wait()

    return gather_kernel


def kernel(text, weight):
    b, l = text.shape
    vocab, embed = weight.shape
    idx = text.T.astype(jnp.int32)
    out5d = _build(b, l, vocab, embed)(idx, weight)
    return out5d.transpose(2, 4, 0, 1, 3).reshape(b, l, embed)


# speedup vs baseline: 1.0524x; 1.0524x over previous
import functools

import jax
import jax.numpy as jnp
from jax import lax
from jax.experimental import pallas as pl
from jax.experimental.pallas import tpu as pltpu
from jax.experimental.pallas import tpu_sc as plsc

NC = 2
NS = 16
NW = NC * NS

LANES = 16
BB = 128
DEPTH = 8


@functools.lru_cache(maxsize=None)
def _build(b, l, vocab, embed):
    nbb = b // BB
    bb_per_w = nbb // NW
    b_per_w = b // NW
    nrb = embed // 8
    units = l * bb_per_w

    mesh = plsc.VectorSubcoreMesh(core_axis_name="c", subcore_axis_name="s")

    @functools.partial(
        pl.kernel,
        out_type=jax.ShapeDtypeStruct((l, nrb, nbb, 8, BB), jnp.float32),
        mesh=mesh,
        scratch_types=[
            pltpu.VMEM((l, b_per_w), jnp.int32),
            pltpu.VMEM((DEPTH, BB, embed), jnp.float32),
            pltpu.VMEM((DEPTH, nrb, 8, BB), jnp.float32),
        ] + [pltpu.SemaphoreType.DMA] * (2 * DEPTH),
        compiler_params=pltpu.CompilerParams(use_tc_tiling_on_sc=False, needs_layout_passes=False),
    )
    def gather_kernel(idx_hbm, table_hbm, out_hbm, idx_v, rows_v, stg_v,
                      *sems):
        gsems = sems[:DEPTH]
        ssems = sems[DEPTH:]
        wid = lax.axis_index("s") * NC + lax.axis_index("c")
        row16s = [lax.iota(jnp.int32, LANES) + m * LANES
                  for m in range(BB // LANES)]
        pltpu.sync_copy(idx_hbm.at[:, pl.ds(wid * b_per_w, b_per_w)], idx_v)

        def unit_lcb(u):
            return u // bb_per_w, lax.rem(u, bb_per_w)

        def gather_descr(u, bufset):
            ul, ucb = unit_lcb(u)
            return pltpu.make_async_copy(
                table_hbm.at[idx_v.at[ul, pl.ds(ucb * BB, BB)]],
                rows_v.at[bufset],
                gsems[bufset],
            )

        def store_descr(u, bufset, rb):
            ul, ucb = unit_lcb(u)
            return pltpu.make_async_copy(
                stg_v.at[bufset, rb],
                out_hbm.at[ul, rb, wid * bb_per_w + ucb],
                ssems[bufset],
            )

        def transpose_unit(bufset):
            src = rows_v.at[bufset]
            for rb in range(nrb):
                for s in range(8):
                    col16 = jnp.full((LANES,), rb * 8 + s, jnp.int32)
                    for m in range(BB // LANES):
                        val = plsc.load_gather(src, [row16s[m], col16])
                        stg_v[bufset, rb, s, pl.ds(m * LANES, LANES)] = val

        for u0 in range(DEPTH - 1):
            gather_descr(u0, u0).start()

        def body(i, _):
            for sub in range(DEPTH):
                u = DEPTH * i + sub
                bufset = sub

                @pl.when(u + DEPTH - 1 < units)
                def _():
                    gather_descr(u + DEPTH - 1, (sub + DEPTH - 1) % DEPTH).start()

                gather_descr(u, bufset).wait()

                @pl.when(u >= DEPTH)
                def _():
                    for rb in range(nrb):
                        store_descr(u - DEPTH, bufset, rb).wait()

                transpose_unit(bufset)
                for rb in range(nrb):
                    store_descr(u, bufset, rb).start()
            return 0

        lax.fori_loop(0, units // DEPTH, body, 0)
        for u in range(units - DEPTH, units):
            for rb in range(nrb):
                store_descr(u, u % DEPTH, rb).wait()

    return gather_kernel


def kernel(text, weight):
    b, l = text.shape
    vocab, embed = weight.shape
    idx = text.T.astype(jnp.int32)
    out5d = _build(b, l, vocab, embed)(idx, weight)
    return out5d.transpose(2, 4, 0, 1, 3).reshape(b, l, embed)


# speedup vs baseline: 1.0978x; 1.0431x over previous
import functools

import jax
import jax.numpy as jnp
from jax import lax
from jax.experimental import pallas as pl
from jax.experimental.pallas import tpu as pltpu
from jax.experimental.pallas import tpu_sc as plsc

NC = 2
NS = 16
NW = NC * NS

LANES = 16
BB = 128
DEPTH = 8


@functools.lru_cache(maxsize=None)
def _build(b, l, vocab, embed):
    nbb = b // BB
    bb_per_w = nbb // NW
    b_per_w = b // NW
    nrb = embed // 8
    units = l * bb_per_w

    mesh = plsc.VectorSubcoreMesh(core_axis_name="c", subcore_axis_name="s")

    @functools.partial(
        pl.kernel,
        out_type=jax.ShapeDtypeStruct((l, nrb, nbb, 8, BB), jnp.float32),
        mesh=mesh,
        scratch_types=[
            pltpu.VMEM((l, b_per_w), jnp.int32),
            pltpu.VMEM((DEPTH, BB, embed), jnp.float32),
            pltpu.VMEM((DEPTH, nrb, 8, BB), jnp.float32),
        ] + [pltpu.SemaphoreType.DMA] * (2 * DEPTH),
        compiler_params=pltpu.CompilerParams(use_tc_tiling_on_sc=False, needs_layout_passes=False),
    )
    def gather_kernel(idx_hbm, table_hbm, out_hbm, idx_v, rows_v, stg_v,
                      *sems):
        gsems = sems[:DEPTH]
        ssems = sems[DEPTH:]
        wid = lax.axis_index("s") * NC + lax.axis_index("c")
        row16s = [lax.iota(jnp.int32, LANES) + m * LANES
                  for m in range(BB // LANES)]
        pltpu.sync_copy(idx_hbm.at[:, pl.ds(wid * b_per_w, b_per_w)], idx_v)

        def unit_lcb(u):
            return u // bb_per_w, lax.rem(u, bb_per_w)

        def gather_descr(u, bufset):
            ul, ucb = unit_lcb(u)
            return pltpu.make_async_copy(
                table_hbm.at[idx_v.at[ul, pl.ds(ucb * BB, BB)]],
                rows_v.at[bufset],
                gsems[bufset],
            )

        def store_descr(u, bufset, rb):
            ul, ucb = unit_lcb(u)
            return pltpu.make_async_copy(
                stg_v.at[bufset, rb],
                out_hbm.at[ul, rb, wid * bb_per_w + ucb],
                ssems[bufset],
            )

        def transpose_unit(bufset):
            src = rows_v.at[bufset]
            for rb in range(nrb):
                for sh in range(2):
                    vals = []
                    for s in range(4 * sh, 4 * sh + 4):
                        col16 = jnp.full((LANES,), rb * 8 + s, jnp.int32)
                        for m in range(BB // LANES):
                            vals.append(
                                (s, m, plsc.load_gather(src, [row16s[m], col16]))
                            )
                    for s, m, val in vals:
                        stg_v[bufset, rb, s, pl.ds(m * LANES, LANES)] = val

        for u0 in range(DEPTH - 1):
            gather_descr(u0, u0).start()

        def body(i, _):
            for sub in range(DEPTH):
                u = DEPTH * i + sub
                bufset = sub

                @pl.when(u + DEPTH - 1 < units)
                def _():
                    gather_descr(u + DEPTH - 1, (sub + DEPTH - 1) % DEPTH).start()

                gather_descr(u, bufset).wait()

                @pl.when(u >= DEPTH)
                def _():
                    for rb in range(nrb):
                        store_descr(u - DEPTH, bufset, rb).wait()

                transpose_unit(bufset)
                for rb in range(nrb):
                    store_descr(u, bufset, rb).start()
            return 0

        lax.fori_loop(0, units // DEPTH, body, 0)
        for u in range(units - DEPTH, units):
            for rb in range(nrb):
                store_descr(u, u % DEPTH, rb).wait()

    return gather_kernel


def kernel(text, weight):
    b, l = text.shape
    vocab, embed = weight.shape
    idx = text.T.astype(jnp.int32)
    out5d = _build(b, l, vocab, embed)(idx, weight)
    return out5d.transpose(2, 4, 0, 1, 3).reshape(b, l, embed)
